# baseline (speedup 1.0000x reference)
import functools

import jax
import jax.numpy as jnp
from jax import lax
from jax.experimental import pallas as pl
from jax.experimental.pallas import tpu as pltpu
from jax.experimental.pallas import tpu_sc as plsc

B = 16384
N_BINS = 64
NC = 2
NS = 16
NW = NC * NS
CHUNK = B // NW
L = 16
NVEC = CHUNK // L
NTBL = N_BINS // L

_mesh = plsc.VectorSubcoreMesh(core_axis_name="c", subcore_axis_name="s")


@functools.partial(
    pl.kernel,
    mesh=_mesh,
    compiler_params=pltpu.CompilerParams(needs_layout_passes=False),
    out_type=[
        jax.ShapeDtypeStruct((B,), jnp.float32),
        jax.ShapeDtypeStruct((B,), jnp.float32),
    ],
    scratch_types=[
        pltpu.VMEM((N_BINS,), jnp.float32),
        pltpu.VMEM((N_BINS,), jnp.float32),
        pltpu.VMEM((N_BINS,), jnp.float32),
        pltpu.VMEM((N_BINS,), jnp.float32),
        pltpu.VMEM((N_BINS,), jnp.float32),
        pltpu.VMEM((CHUNK,), jnp.int32),
        pltpu.VMEM((CHUNK,), jnp.float32),
        pltpu.VMEM((CHUNK,), jnp.float32),
        pltpu.VMEM((CHUNK,), jnp.float32),
        pltpu.SemaphoreType.DMA,
        pltpu.SemaphoreType.DMA,
        pltpu.SemaphoreType.DMA,
        pltpu.SemaphoreType.DMA,
        pltpu.SemaphoreType.DMA,
        pltpu.SemaphoreType.DMA,
        pltpu.SemaphoreType.DMA,
    ],
)
def _hazard_sc(logw_hbm, bp_hbm, w_hbm, ts_hbm, t_hbm,
               emb_hbm, ch_hbm,
               logw_v, bp_v, w_v, a_v, c_v, ts_v, t_v, emb_v, ch_v,
               sem_lw, sem_bp, sem_w, sem_ts, sem_t, sem_emb, sem_ch):
    wid = lax.axis_index("s") * NC + lax.axis_index("c")
    base = wid * CHUNK

    cp_lw = pltpu.async_copy(logw_hbm, logw_v, sem_lw)
    cp_bp = pltpu.async_copy(bp_hbm, bp_v, sem_bp)
    cp_w = pltpu.async_copy(w_hbm, w_v, sem_w)
    cp_ts = pltpu.async_copy(ts_hbm.at[pl.ds(base, CHUNK)], ts_v, sem_ts)
    cp_t = pltpu.async_copy(t_hbm.at[pl.ds(base, CHUNK)], t_v, sem_t)
    cp_lw.wait()
    cp_bp.wait()
    cp_w.wait()

    carry = jnp.float32(0.0)
    for j in range(NTBL):
        sl = pl.ds(j * L, L)
        lam = jnp.exp(logw_v[sl])
        aw = lam * w_v[sl]
        incl = jnp.cumsum(aw)
        a_v[sl] = lam
        c_v[sl] = (incl - aw + carry) - lam * bp_v[sl]
        carry = carry + incl[L - 1]

    cp_ts.wait()
    cp_t.wait()

    for i in range(NVEC):
        sl = pl.ds(i * L, L)
        idx = ts_v[sl]
        emb_v[sl] = plsc.load_gather(logw_v, [idx])
        a = plsc.load_gather(a_v, [idx])
        c = plsc.load_gather(c_v, [idx])
        ch_v[sl] = a * t_v[sl] + c

    cp_emb = pltpu.async_copy(emb_v, emb_hbm.at[pl.ds(base, CHUNK)], sem_emb)
    cp_ch = pltpu.async_copy(ch_v, ch_hbm.at[pl.ds(base, CHUNK)], sem_ch)
    cp_emb.wait()
    cp_ch.wait()


def kernel(x, t, t_section, logw, breakpoints, widths):
    del x
    emb, ch = _hazard_sc(
        logw.reshape(N_BINS),
        breakpoints.reshape(N_BINS),
        widths.reshape(N_BINS),
        t_section.astype(jnp.int32),
        t.reshape(B),
    )
    return emb.reshape(B, 1), ch.reshape(B, 1)


# speedup vs baseline: 1.0402x; 1.0402x over previous
import functools

import jax
import jax.numpy as jnp
from jax import lax
from jax.experimental import pallas as pl
from jax.experimental.pallas import tpu as pltpu
from jax.experimental.pallas import tpu_sc as plsc

B = 16384
N_BINS = 64
NC = 1
NS = 16
NW = NC * NS
CHUNK = B // NW
L = 16
NVEC = CHUNK // L
NTBL = N_BINS // L

_mesh = plsc.VectorSubcoreMesh(core_axis_name="c", subcore_axis_name="s", num_cores=1)


@functools.partial(
    pl.kernel,
    mesh=_mesh,
    compiler_params=pltpu.CompilerParams(needs_layout_passes=False),
    out_type=[
        jax.ShapeDtypeStruct((B,), jnp.float32),
        jax.ShapeDtypeStruct((B,), jnp.float32),
    ],
    scratch_types=[
        pltpu.VMEM((N_BINS,), jnp.float32),
        pltpu.VMEM((N_BINS,), jnp.float32),
        pltpu.VMEM((N_BINS,), jnp.float32),
        pltpu.VMEM((N_BINS,), jnp.float32),
        pltpu.VMEM((N_BINS,), jnp.float32),
        pltpu.VMEM((CHUNK,), jnp.int32),
        pltpu.VMEM((CHUNK,), jnp.float32),
        pltpu.VMEM((CHUNK,), jnp.float32),
        pltpu.VMEM((CHUNK,), jnp.float32),
        pltpu.SemaphoreType.DMA,
        pltpu.SemaphoreType.DMA,
        pltpu.SemaphoreType.DMA,
        pltpu.SemaphoreType.DMA,
        pltpu.SemaphoreType.DMA,
        pltpu.SemaphoreType.DMA,
        pltpu.SemaphoreType.DMA,
    ],
)
def _hazard_sc(logw_hbm, bp_hbm, w_hbm, ts_hbm, t_hbm,
               emb_hbm, ch_hbm,
               logw_v, bp_v, w_v, a_v, c_v, ts_v, t_v, emb_v, ch_v,
               sem_lw, sem_bp, sem_w, sem_ts, sem_t, sem_emb, sem_ch):
    wid = lax.axis_index("s") * NC + lax.axis_index("c")
    base = wid * CHUNK

    cp_lw = pltpu.async_copy(logw_hbm, logw_v, sem_lw)
    cp_bp = pltpu.async_copy(bp_hbm, bp_v, sem_bp)
    cp_w = pltpu.async_copy(w_hbm, w_v, sem_w)
    cp_ts = pltpu.async_copy(ts_hbm.at[pl.ds(base, CHUNK)], ts_v, sem_ts)
    cp_t = pltpu.async_copy(t_hbm.at[pl.ds(base, CHUNK)], t_v, sem_t)
    cp_lw.wait()
    cp_bp.wait()
    cp_w.wait()

    carry = jnp.float32(0.0)
    for j in range(NTBL):
        sl = pl.ds(j * L, L)
        lam = jnp.exp(logw_v[sl])
        aw = lam * w_v[sl]
        incl = jnp.cumsum(aw)
        a_v[sl] = lam
        c_v[sl] = (incl - aw + carry) - lam * bp_v[sl]
        carry = carry + incl[L - 1]

    cp_ts.wait()
    cp_t.wait()

    for i in range(NVEC):
        sl = pl.ds(i * L, L)
        idx = ts_v[sl]
        emb_v[sl] = plsc.load_gather(logw_v, [idx])
        a = plsc.load_gather(a_v, [idx])
        c = plsc.load_gather(c_v, [idx])
        ch_v[sl] = a * t_v[sl] + c

    cp_emb = pltpu.async_copy(emb_v, emb_hbm.at[pl.ds(base, CHUNK)], sem_emb)
    cp_ch = pltpu.async_copy(ch_v, ch_hbm.at[pl.ds(base, CHUNK)], sem_ch)
    cp_emb.wait()
    cp_ch.wait()


def kernel(x, t, t_section, logw, breakpoints, widths):
    del x
    emb, ch = _hazard_sc(
        logw.reshape(N_BINS),
        breakpoints.reshape(N_BINS),
        widths.reshape(N_BINS),
        t_section.astype(jnp.int32),
        t.reshape(B),
    )
    return emb.reshape(B, 1), ch.reshape(B, 1)


# speedup vs baseline: 1.0984x; 1.0559x over previous
import functools

import jax
import jax.numpy as jnp
from jax import lax
from jax.experimental import pallas as pl
from jax.experimental.pallas import tpu as pltpu
from jax.experimental.pallas import tpu_sc as plsc

B = 16384
N_BINS = 64
NC = 1
NS = 16
NW = NC * NS
CHUNK = B // NW
L = 16
NVEC = CHUNK // L
NTBL = N_BINS // L

_mesh = plsc.VectorSubcoreMesh(core_axis_name="c", subcore_axis_name="s", num_cores=1)


@functools.partial(
    pl.kernel,
    mesh=_mesh,
    compiler_params=pltpu.CompilerParams(needs_layout_passes=False),
    out_type=[
        jax.ShapeDtypeStruct((B,), jnp.float32),
        jax.ShapeDtypeStruct((B,), jnp.float32),
    ],
    scratch_types=[
        pltpu.VMEM((N_BINS,), jnp.float32),
        pltpu.VMEM((N_BINS,), jnp.float32),
        pltpu.VMEM((N_BINS,), jnp.float32),
        pltpu.VMEM((N_BINS,), jnp.float32),
        pltpu.VMEM((N_BINS,), jnp.float32),
        pltpu.VMEM((CHUNK,), jnp.int32),
        pltpu.VMEM((CHUNK,), jnp.float32),
        pltpu.VMEM((CHUNK,), jnp.float32),
        pltpu.VMEM((CHUNK,), jnp.float32),
        pltpu.SemaphoreType.DMA,
        pltpu.SemaphoreType.DMA,
        pltpu.SemaphoreType.DMA,
        pltpu.SemaphoreType.DMA,
        pltpu.SemaphoreType.DMA,
        pltpu.SemaphoreType.DMA,
        pltpu.SemaphoreType.DMA,
    ],
)
def _hazard_sc(logw_hbm, bp_hbm, w_hbm, ts_hbm, t_hbm,
               emb_hbm, ch_hbm,
               logw_v, bp_v, w_v, a_v, c_v, ts_v, t_v, emb_v, ch_v,
               sem_lw, sem_bp, sem_w, sem_ts, sem_t, sem_emb, sem_ch):
    wid = lax.axis_index("s") * NC + lax.axis_index("c")
    base = wid * CHUNK

    cp_lw = pltpu.async_copy(logw_hbm, logw_v, sem_lw)
    cp_bp = pltpu.async_copy(bp_hbm, bp_v, sem_bp)
    cp_w = pltpu.async_copy(w_hbm, w_v, sem_w)
    cp_ts = pltpu.async_copy(ts_hbm.at[pl.ds(base, CHUNK)], ts_v, sem_ts)
    cp_t = pltpu.async_copy(t_hbm.at[pl.ds(base, CHUNK)], t_v, sem_t)
    cp_lw.wait()
    cp_bp.wait()
    cp_w.wait()

    carry = jnp.float32(0.0)
    for j in range(NTBL):
        sl = pl.ds(j * L, L)
        lam = jnp.exp(logw_v[sl])
        aw = lam * w_v[sl]
        incl = jnp.cumsum(aw)
        a_v[sl] = lam
        c_v[sl] = (incl - aw + carry) - lam * bp_v[sl]
        carry = carry + incl[L - 1]

    cp_ts.wait()
    cp_t.wait()

    def body(i, _):
        sl = pl.ds(i * L, L)
        idx = ts_v[sl]
        emb_v[sl] = plsc.load_gather(logw_v, [idx])
        a = plsc.load_gather(a_v, [idx])
        c = plsc.load_gather(c_v, [idx])
        ch_v[sl] = a * t_v[sl] + c
        return 0

    lax.fori_loop(0, NVEC, body, 0)

    cp_emb = pltpu.async_copy(emb_v, emb_hbm.at[pl.ds(base, CHUNK)], sem_emb)
    cp_ch = pltpu.async_copy(ch_v, ch_hbm.at[pl.ds(base, CHUNK)], sem_ch)
    cp_emb.wait()
    cp_ch.wait()


def kernel(x, t, t_section, logw, breakpoints, widths):
    del x
    emb, ch = _hazard_sc(
        logw.reshape(N_BINS),
        breakpoints.reshape(N_BINS),
        widths.reshape(N_BINS),
        t_section.astype(jnp.int32),
        t.reshape(B),
    )
    return emb.reshape(B, 1), ch.reshape(B, 1)


# speedup vs baseline: 1.1081x; 1.0089x over previous
import functools

import jax
import jax.numpy as jnp
from jax import lax
from jax.experimental import pallas as pl
from jax.experimental.pallas import tpu as pltpu
from jax.experimental.pallas import tpu_sc as plsc

B = 16384
N_BINS = 64
NC = 1
NS = 16
NW = NC * NS
CHUNK = B // NW
L = 16
NTBL = N_BINS // L

_mesh = plsc.VectorSubcoreMesh(core_axis_name="c", subcore_axis_name="s", num_cores=1)


@functools.partial(
    pl.kernel,
    mesh=_mesh,
    compiler_params=pltpu.CompilerParams(needs_layout_passes=False),
    out_type=[
        jax.ShapeDtypeStruct((B,), jnp.float32),
        jax.ShapeDtypeStruct((B,), jnp.float32),
    ],
    scratch_types=[
        pltpu.VMEM((3 * N_BINS,), jnp.float32),
        pltpu.VMEM((N_BINS,), jnp.float32),
        pltpu.VMEM((N_BINS,), jnp.float32),
        pltpu.VMEM((2, CHUNK), jnp.int32),
        pltpu.VMEM((CHUNK,), jnp.float32),
        pltpu.VMEM((CHUNK,), jnp.float32),
        pltpu.SemaphoreType.DMA,
        pltpu.SemaphoreType.DMA,
        pltpu.SemaphoreType.DMA,
        pltpu.SemaphoreType.DMA,
    ],
)
def _hazard_sc(tbl_hbm, tst_hbm,
               emb_hbm, ch_hbm,
               tbl_v, a_v, c_v, tst_v, emb_v, ch_v,
               sem_tbl, sem_tst, sem_emb, sem_ch):
    wid = lax.axis_index("s") * NC + lax.axis_index("c")
    base = wid * CHUNK

    cp_tbl = pltpu.async_copy(tbl_hbm, tbl_v, sem_tbl)
    cp_tst = pltpu.async_copy(tst_hbm.at[:, pl.ds(base, CHUNK)], tst_v, sem_tst)
    cp_tbl.wait()

    carry = jnp.float32(0.0)
    for j in range(NTBL):
        sl = pl.ds(j * L, L)
        lam = jnp.exp(tbl_v[sl])
        aw = lam * tbl_v[pl.ds(2 * N_BINS + j * L, L)]
        incl = jnp.cumsum(aw)
        a_v[sl] = lam
        c_v[sl] = (incl - aw + carry) - lam * tbl_v[pl.ds(N_BINS + j * L, L)]
        carry = carry + incl[L - 1]

    cp_tst.wait()

    @plsc.parallel_loop(0, CHUNK, L, unroll=4)
    def _(i):
        sl = pl.ds(i, L)
        idx = tst_v[0, sl]
        t = plsc.bitcast(tst_v[1, sl], jnp.float32)
        emb_v[sl] = plsc.load_gather(tbl_v, [idx])
        a = plsc.load_gather(a_v, [idx])
        c = plsc.load_gather(c_v, [idx])
        ch_v[sl] = a * t + c

    cp_emb = pltpu.async_copy(emb_v, emb_hbm.at[pl.ds(base, CHUNK)], sem_emb)
    cp_ch = pltpu.async_copy(ch_v, ch_hbm.at[pl.ds(base, CHUNK)], sem_ch)
    cp_emb.wait()
    cp_ch.wait()


def kernel(x, t, t_section, logw, breakpoints, widths):
    del x
    tbl = jnp.concatenate(
        [logw.reshape(N_BINS), breakpoints.reshape(N_BINS), widths.reshape(N_BINS)]
    )
    tst = jnp.stack(
        [t_section.astype(jnp.int32),
         lax.bitcast_convert_type(t.reshape(B), jnp.int32)]
    )
    emb, ch = _hazard_sc(tbl, tst)
    return emb.reshape(B, 1), ch.reshape(B, 1)
